# baseline (speedup 1.0000x reference)
import functools
import jax
import jax.numpy as jnp
from jax import lax
from jax.experimental import pallas as pl
from jax.experimental.pallas import tpu as pltpu
from jax.experimental.pallas import tpu_sc as plsc

N = 4096
D = 512
NW = 32
RPW = N // NW
CH = 32
NCH = RPW // CH
NJ = D // 16


def _sc_body(adj_hbm, q_hbm, uni_hbm, sub_hbm, orig_hbm,
             out_sums_hbm, out_head_hbm,
             qv, ablk, mvals, ubuf, sbuf, acc, tmp, sem_u, sem_s):
    nc = 2
    c = lax.axis_index("c")
    s = lax.axis_index("s")
    wid = s * nc + c
    base = wid * RPW

    pltpu.sync_copy(q_hbm, qv)
    q = qv[pl.ds(0, 16)][0]
    col0 = pl.multiple_of((q // 128) * 128, 128)
    lane = q - col0

    pltpu.sync_copy(adj_hbm.at[pl.ds(base, RPW), pl.ds(col0, 128)], ablk)
    lane_vec = jnp.full((16,), lane, jnp.int32)
    for t in range(RPW // 16):
        rid = jax.lax.iota(jnp.int32, 16) + (t * 16)
        vals = plsc.load_gather(ablk, [rid, lane_vec])
        mvals[pl.ds(t * 16, 16)] = jnp.where(vals > 0.0,
                                             jnp.ones((16,), jnp.float32),
                                             jnp.zeros((16,), jnp.float32))

    zero16 = jnp.zeros((16,), jnp.float32)
    for r in range(3):
        for j in range(NJ):
            acc[r, pl.ds(j * 16, 16)] = zero16

    cps_u = [None] * NCH
    cps_s = [None] * NCH
    cps_u[0] = pltpu.async_copy(
        uni_hbm.at[pl.ds(base, CH)], ubuf.at[0], sem_u)
    cps_s[0] = pltpu.async_copy(
        sub_hbm.at[pl.ds(base, CH)], sbuf.at[0], sem_s)

    for ci in range(NCH):
        pb = ci % 2
        if ci + 1 < NCH:
            cps_u[ci + 1] = pltpu.async_copy(
                uni_hbm.at[pl.ds(base + (ci + 1) * CH, CH)],
                ubuf.at[1 - pb], sem_u)
            cps_s[ci + 1] = pltpu.async_copy(
                sub_hbm.at[pl.ds(base + (ci + 1) * CH, CH)],
                sbuf.at[1 - pb], sem_s)
        cps_u[ci].wait()
        cps_s[ci].wait()

        ub = ubuf.at[pb]
        sb = sbuf.at[pb]
        cbase = ci * CH

        def jbody(j, _, ub=ub, sb=sb, cbase=cbase):
            au = jnp.zeros((16,), jnp.float32)
            an = jnp.zeros((16,), jnp.float32)
            asub = jnp.zeros((16,), jnp.float32)
            mgroups = [mvals[pl.ds(cbase + g * 16, 16)] for g in range(CH // 16)]
            for r in range(CH):
                uvec = ub[r, pl.ds(j * 16, 16)]
                svec = sb[r, pl.ds(j * 16, 16)]
                m = mgroups[r // 16][r % 16]
                au = au + uvec
                asub = asub + svec
                an = an + m * uvec
            plsc.addupdate(acc.at[0, pl.ds(j * 16, 16)], an)
            plsc.addupdate(acc.at[1, pl.ds(j * 16, 16)], au)
            plsc.addupdate(acc.at[2, pl.ds(j * 16, 16)], asub)
            return _

        lax.fori_loop(0, NJ, jbody, None)

    pltpu.sync_copy(acc, out_sums_hbm.at[wid])

    @pl.when(wid == 0)
    def _head():
        pltpu.sync_copy(orig_hbm, tmp)
        pltpu.sync_copy(tmp, out_head_hbm.at[pl.ds(0, 1)])
        pltpu.sync_copy(sub_hbm.at[pl.ds(q, 1)], tmp)
        pltpu.sync_copy(tmp, out_head_hbm.at[pl.ds(1, 1)])


def kernel(adj, cur_sub_idx, uni_feat, sub_feat, original_sub_feat):
    qarr = jnp.full((16,), cur_sub_idx, jnp.int32)
    orig = original_sub_feat.reshape((1, D))
    mesh = plsc.VectorSubcoreMesh(core_axis_name="c", subcore_axis_name="s")
    f = functools.partial(
        pl.kernel,
        mesh=mesh,
        compiler_params=pltpu.CompilerParams(needs_layout_passes=False),
        out_type=(
            jax.ShapeDtypeStruct((NW, 3, D), jnp.float32),
            jax.ShapeDtypeStruct((2, D), jnp.float32),
        ),
        scratch_types=[
            pltpu.VMEM((16,), jnp.int32),
            pltpu.VMEM((RPW, 128), jnp.float32),
            pltpu.VMEM((RPW,), jnp.float32),
            pltpu.VMEM((2, CH, D), jnp.float32),
            pltpu.VMEM((2, CH, D), jnp.float32),
            pltpu.VMEM((3, D), jnp.float32),
            pltpu.VMEM((1, D), jnp.float32),
            pltpu.SemaphoreType.DMA,
            pltpu.SemaphoreType.DMA,
        ],
    )(_sc_body)
    out_sums, out_head = f(adj, qarr, uni_feat, sub_feat, orig)
    sums = jnp.sum(out_sums, axis=0)
    return jnp.concatenate(
        (out_head[0], out_head[1], sums[0], sums[1], sums[2]))
